# baseline (speedup 1.0000x reference)
import functools

import jax
import jax.numpy as jnp
from jax import lax
from jax.experimental import pallas as pl
from jax.experimental.pallas import tpu as pltpu
from jax.experimental.pallas import tpu_sc as plsc

_DIM = 64
_SEQ = 200
_NC = 2
_NS = 16
_NW = _NC * _NS
_L = 16


def _emb_mean_body(bpw, textT_hbm, table_hbm, out_hbm, idx_v, rows_v, acc_v,
                   sem_a, sem_b):
    wid = lax.axis_index("s") * _NC + lax.axis_index("c")
    b0 = wid * bpw

    pltpu.sync_copy(textT_hbm.at[:, pl.ds(b0, bpw)], idx_v)

    def zbody(r, _):
        for k in range(_DIM // _L):
            acc_v[r, pl.ds(k * _L, _L)] = jnp.zeros((_L,), jnp.float32)
        return 0

    lax.fori_loop(0, bpw, zbody, 0)

    buf_a = rows_v.at[0]
    buf_b = rows_v.at[1]

    def fire(l, buf, sem):
        pltpu.async_copy(table_hbm.at[idx_v.at[l]], buf, sem)

    def drain(buf, sem):
        pltpu.make_async_copy(table_hbm.at[pl.ds(0, bpw)], buf, sem).wait()

    def accum(buf):
        def rbody(g, _):
            for u in range(2):
                r = g * 2 + u
                for k in range(_DIM // _L):
                    plsc.addupdate(acc_v.at[r, pl.ds(k * _L, _L)],
                                   buf[r, pl.ds(k * _L, _L)])
            return 0

        lax.fori_loop(0, bpw // 2, rbody, 0)

    fire(0, buf_a, sem_a)

    def body(j, _):
        l = j * 2
        fire(l + 1, buf_b, sem_b)
        drain(buf_a, sem_a)
        accum(buf_a)

        @pl.when(l + 2 < _SEQ)
        def _():
            fire(l + 2, buf_a, sem_a)

        drain(buf_b, sem_b)
        accum(buf_b)
        return 0

    lax.fori_loop(0, _SEQ // 2, body, 0)

    scale = jnp.float32(1.0 / _SEQ)

    def sbody(r, _):
        for k in range(_DIM // _L):
            acc_v[r, pl.ds(k * _L, _L)] = acc_v[r, pl.ds(k * _L, _L)] * scale
        return 0

    lax.fori_loop(0, bpw, sbody, 0)
    pltpu.sync_copy(acc_v, out_hbm.at[pl.ds(b0, bpw)])


@functools.partial(jax.jit, static_argnames=("batch",))
def _emb_mean(textT, table, batch):
    bpw = batch // _NW
    mesh = plsc.VectorSubcoreMesh(
        core_axis_name="c", subcore_axis_name="s",
        num_cores=_NC, num_subcores=_NS)
    return pl.kernel(
        functools.partial(_emb_mean_body, bpw),
        out_type=jax.ShapeDtypeStruct((batch, _DIM), jnp.float32),
        mesh=mesh,
        compiler_params=pltpu.CompilerParams(use_tc_tiling_on_sc=False),
        scratch_types=[
            pltpu.VMEM((_SEQ, bpw), jnp.int32),
            pltpu.VMEM((2, bpw, _DIM), jnp.float32),
            pltpu.VMEM((bpw, _DIM), jnp.float32),
            pltpu.SemaphoreType.DMA,
            pltpu.SemaphoreType.DMA,
        ],
    )(textT, table)


def kernel(text, table):
    batch = text.shape[0]
    textT = jnp.swapaxes(text.astype(jnp.int32), 0, 1)
    return _emb_mean(textT, table, batch)


# speedup vs baseline: 1.1275x; 1.1275x over previous
import functools

import jax
import jax.numpy as jnp
from jax import lax
from jax.experimental import pallas as pl
from jax.experimental.pallas import tpu as pltpu
from jax.experimental.pallas import tpu_sc as plsc

_DIM = 64
_SEQ = 200
_NC = 2
_NS = 16
_NW = _NC * _NS
_L = 16
_TR = _SEQ // 8


def _mesh():
    return plsc.VectorSubcoreMesh(
        core_axis_name="c", subcore_axis_name="s",
        num_cores=_NC, num_subcores=_NS)


def _wid():
    return lax.axis_index("s") * _NC + lax.axis_index("c")


def _detile_body(textT_hbm, idx_hbm, stage_v, sem):
    wid = _wid()

    cps = [
        pltpu.async_copy(
            textT_hbm.at[pl.ds(i * 8, 8), pl.ds(wid * 128, 128)],
            stage_v.at[i], sem)
        for i in range(_TR)
    ]
    for cp in cps:
        cp.wait()
    pltpu.sync_copy(stage_v, idx_hbm.at[pl.ds(wid * _TR, _TR)])


def _emb_mean_body(bpw, idx_hbm, table_hbm, out_hbm, idx_v, acc_v, sem):
    wid = _wid()
    b0 = wid * bpw

    pltpu.sync_copy(idx_hbm.at[pl.ds(wid * _TR, _TR)], idx_v)

    def zbody(r, _):
        for k in range(_DIM // _L):
            acc_v[r, pl.ds(k * _L, _L)] = jnp.zeros((_L,), jnp.float32)
        return 0

    lax.fori_loop(0, bpw, zbody, 0)

    def fire(i):
        for r in range(8):
            pltpu.async_copy(table_hbm.at[idx_v.at[i, r]], acc_v, sem,
                             add=True)

    def drain():
        for _ in range(8):
            pltpu.make_async_copy(table_hbm.at[pl.ds(0, bpw)], acc_v,
                                  sem).wait()

    fire(0)

    def body(i, _):
        fire(i)
        drain()
        return 0

    lax.fori_loop(1, _TR, body, 0)
    drain()

    scale = jnp.float32(1.0 / _SEQ)

    def sbody(r, _):
        for k in range(_DIM // _L):
            acc_v[r, pl.ds(k * _L, _L)] = acc_v[r, pl.ds(k * _L, _L)] * scale
        return 0

    lax.fori_loop(0, bpw, sbody, 0)
    pltpu.sync_copy(acc_v, out_hbm.at[pl.ds(b0, bpw)])


@functools.partial(jax.jit, static_argnames=("batch",))
def _emb_mean(textT, table, batch):
    bpw = batch // _NW
    idx = pl.kernel(
        _detile_body,
        out_type=jax.ShapeDtypeStruct((_NW * _TR, 8, 128), jnp.int32),
        mesh=_mesh(),
        compiler_params=pltpu.CompilerParams(use_tc_tiling_on_sc=True),
        scratch_types=[
            pltpu.VMEM((_TR, 8, 128), jnp.int32),
            pltpu.SemaphoreType.DMA,
        ],
    )(textT)
    return pl.kernel(
        functools.partial(_emb_mean_body, bpw),
        out_type=jax.ShapeDtypeStruct((batch, _DIM), jnp.float32),
        mesh=_mesh(),
        compiler_params=pltpu.CompilerParams(use_tc_tiling_on_sc=False),
        scratch_types=[
            pltpu.VMEM((_TR, 8, 128), jnp.int32),
            pltpu.VMEM((bpw, _DIM), jnp.float32),
            pltpu.SemaphoreType.DMA,
        ],
    )(idx, table)


def kernel(text, table):
    batch = text.shape[0]
    textT = jnp.swapaxes(text.astype(jnp.int32), 0, 1)
    return _emb_mean(textT, table, batch)
